# Initial kernel scaffold; baseline (speedup 1.0000x reference)
import functools

import jax
import jax.numpy as jnp
from jax import lax
from jax.experimental import pallas as pl
from jax.experimental.pallas import tpu as pltpu
from jax.experimental.pallas import tpu_sc as plsc

_LANES = 16


@functools.lru_cache(maxsize=None)
def _build_sc_kernel(n_tokens, n_widx, n_widx_pad, n_pidx, emb_dim):
    info = plsc.get_sparse_core_info()
    num_cores, num_subcores = info.num_cores, info.num_subcores
    n_workers = num_cores * num_subcores
    tok_per_worker = n_tokens // n_workers
    C = 8
    n_chunks = tok_per_worker // C
    n_col = emb_dim // _LANES
    inv128 = 1.0 / (4 * n_pidx)
    inv4 = 0.25

    mesh = plsc.VectorSubcoreMesh(core_axis_name="c", subcore_axis_name="s")

    @functools.partial(
        pl.kernel,
        out_type=jax.ShapeDtypeStruct((n_tokens, emb_dim), jnp.float32),
        mesh=mesh,
        scratch_types=[
            pltpu.VMEM((C, n_widx_pad), jnp.int32),
            pltpu.VMEM((C, n_pidx), jnp.int32),
            pltpu.VMEM((C, n_widx, emb_dim), jnp.float32),
            pltpu.VMEM((C, n_pidx, emb_dim), jnp.float32),
            pltpu.VMEM((C, emb_dim), jnp.float32),
            pltpu.SemaphoreType.DMA,
        ],
    )
    def sc_kernel(widx_hbm, pidx_hbm, word_hbm, postag_hbm, out_hbm,
                  widx_v, pidx_v, wrows_v, prows_v, outbuf_v, sem):
        wid = lax.axis_index("s") * num_cores + lax.axis_index("c")
        base = wid * tok_per_worker

        def chunk_body(ci, _):
            tok0 = base + ci * C
            pltpu.sync_copy(widx_hbm.at[pl.ds(tok0, C)], widx_v)
            pltpu.sync_copy(pidx_hbm.at[pl.ds(tok0, C)], pidx_v)
            cps = []
            for t in range(C):
                cps.append(pltpu.async_copy(
                    word_hbm.at[widx_v.at[t, pl.ds(0, n_widx)]],
                    wrows_v.at[t], sem))
                cps.append(pltpu.async_copy(
                    postag_hbm.at[pidx_v.at[t]],
                    prows_v.at[t], sem))
            for cp in cps:
                cp.wait()

            def tok_body(t, _):
                def wrow_body(r, accs):
                    return [accs[c] + wrows_v[t, r, pl.ds(c * _LANES, _LANES)]
                            for c in range(n_col)]

                def prow_body(r, accs):
                    return [accs[c] + prows_v[t, r, pl.ds(c * _LANES, _LANES)]
                            for c in range(n_col)]

                zero = [jnp.zeros((_LANES,), jnp.float32)] * n_col
                s_all = lax.fori_loop(0, n_widx, wrow_body, zero)
                p_sum = lax.fori_loop(0, n_pidx, prow_body, zero)
                for c in range(n_col):
                    w0 = wrows_v[t, 0, pl.ds(c * _LANES, _LANES)]
                    res = (s_all[c] - w0 + p_sum[c]) * inv128 + w0 * inv4
                    outbuf_v[t, pl.ds(c * _LANES, _LANES)] = res
                return 0

            lax.fori_loop(0, C, tok_body, 0)
            pltpu.sync_copy(outbuf_v, out_hbm.at[pl.ds(tok0, C)])
            return 0

        lax.fori_loop(0, n_chunks, chunk_body, 0)

    return sc_kernel


def kernel(word_idx, form_idx, lemma_idx, postag_idx, word_table, postag_table):
    n = word_idx.shape[0]
    n_morph = form_idx.shape[1] * form_idx.shape[2]
    emb_dim = word_table.shape[1]
    n_widx = 1 + 2 * n_morph
    n_widx_pad = (n_widx + 7) // 8 * 8

    widx = jnp.concatenate(
        [
            word_idx.astype(jnp.int32)[:, None],
            form_idx.reshape(n, n_morph).astype(jnp.int32),
            lemma_idx.reshape(n, n_morph).astype(jnp.int32),
            jnp.zeros((n, n_widx_pad - n_widx), jnp.int32),
        ],
        axis=1,
    )
    pidx = postag_idx.reshape(n, n_morph).astype(jnp.int32)

    sc = _build_sc_kernel(n, n_widx, n_widx_pad, n_morph, emb_dim)
    return sc(widx, pidx, word_table.astype(jnp.float32),
              postag_table.astype(jnp.float32))


# speedup vs baseline: 2.5531x; 2.5531x over previous
import functools

import jax
import jax.numpy as jnp
from jax import lax
from jax.experimental import pallas as pl
from jax.experimental.pallas import tpu as pltpu
from jax.experimental.pallas import tpu_sc as plsc

_LANES = 16


@functools.lru_cache(maxsize=None)
def _build_sc_kernel(n_tokens, n_midx, n_pidx, emb_dim):
    info = plsc.get_sparse_core_info()
    num_cores, num_subcores = info.num_cores, info.num_subcores
    n_workers = num_cores * num_subcores
    tok_per_worker = n_tokens // n_workers
    C = 8
    n_chunks = tok_per_worker // C
    n_col = emb_dim // _LANES
    inv128 = 1.0 / (2 * n_midx)
    inv4 = 0.25

    mesh = plsc.VectorSubcoreMesh(core_axis_name="c", subcore_axis_name="s")

    @functools.partial(
        pl.kernel,
        out_type=jax.ShapeDtypeStruct((n_tokens, emb_dim), jnp.float32),
        mesh=mesh,
        compiler_params=pltpu.CompilerParams(use_tc_tiling_on_sc=False),
        scratch_types=[
            pltpu.VMEM((C, n_midx), jnp.int32),
            pltpu.VMEM((C, n_pidx), jnp.int32),
            pltpu.VMEM((C,), jnp.int32),
            pltpu.VMEM((C, n_midx, emb_dim), jnp.float32),
            pltpu.VMEM((C, n_pidx, emb_dim), jnp.float32),
            pltpu.VMEM((C, emb_dim), jnp.float32),
            pltpu.VMEM((C, emb_dim), jnp.float32),
            pltpu.SemaphoreType.DMA,
        ],
    )
    def sc_kernel(midx_hbm, pidx_hbm, widx_hbm, word_hbm, postag_hbm, out_hbm,
                  midx_v, pidx_v, widx_v, mrows_v, prows_v, wrows_v,
                  outbuf_v, sem):
        wid = lax.axis_index("s") * num_cores + lax.axis_index("c")
        base = wid * tok_per_worker

        def chunk_body(ci, _):
            tok0 = base + ci * C
            pltpu.sync_copy(midx_hbm.at[pl.ds(tok0, C)], midx_v)
            pltpu.sync_copy(pidx_hbm.at[pl.ds(tok0, C)], pidx_v)
            pltpu.sync_copy(widx_hbm.at[pl.ds(tok0, C)], widx_v)
            cps = [pltpu.async_copy(word_hbm.at[widx_v], wrows_v, sem)]
            for t in range(C):
                cps.append(pltpu.async_copy(
                    word_hbm.at[midx_v.at[t]], mrows_v.at[t], sem))
                cps.append(pltpu.async_copy(
                    postag_hbm.at[pidx_v.at[t]], prows_v.at[t], sem))
            for cp in cps:
                cp.wait()

            def tok_body(t, _):
                def mrow_body(r, accs):
                    return [accs[c] + mrows_v[t, r, pl.ds(c * _LANES, _LANES)]
                            for c in range(n_col)]

                def prow_body(r, accs):
                    return [accs[c] + prows_v[t, r, pl.ds(c * _LANES, _LANES)]
                            for c in range(n_col)]

                zero = [jnp.zeros((_LANES,), jnp.float32)] * n_col
                m_sum = lax.fori_loop(0, n_midx, mrow_body, zero)
                p_sum = lax.fori_loop(0, n_pidx, prow_body, zero)
                for c in range(n_col):
                    w = wrows_v[t, pl.ds(c * _LANES, _LANES)]
                    res = (m_sum[c] + p_sum[c]) * inv128 + w * inv4
                    outbuf_v[t, pl.ds(c * _LANES, _LANES)] = res
                return 0

            lax.fori_loop(0, C, tok_body, 0)
            pltpu.sync_copy(outbuf_v, out_hbm.at[pl.ds(tok0, C)])
            return 0

        lax.fori_loop(0, n_chunks, chunk_body, 0)

    return sc_kernel


def kernel(word_idx, form_idx, lemma_idx, postag_idx, word_table, postag_table):
    n = word_idx.shape[0]
    n_morph = form_idx.shape[1] * form_idx.shape[2]
    emb_dim = word_table.shape[1]

    midx = jnp.concatenate(
        [
            form_idx.reshape(n, n_morph).astype(jnp.int32),
            lemma_idx.reshape(n, n_morph).astype(jnp.int32),
        ],
        axis=1,
    )
    pidx = postag_idx.reshape(n, n_morph).astype(jnp.int32)

    sc = _build_sc_kernel(n, 2 * n_morph, n_morph, emb_dim)
    return sc(midx, pidx, word_idx.astype(jnp.int32),
              word_table.astype(jnp.float32), postag_table.astype(jnp.float32))


# speedup vs baseline: 2.8012x; 1.0972x over previous
import functools

import jax
import jax.numpy as jnp
from jax import lax
from jax.experimental import pallas as pl
from jax.experimental.pallas import tpu as pltpu
from jax.experimental.pallas import tpu_sc as plsc

_LANES = 16


@functools.lru_cache(maxsize=None)
def _build_sc_kernel(n_tokens, n_rows, n_morph2, n_pos, emb_dim):
    info = plsc.get_sparse_core_info()
    num_cores, num_subcores = info.num_cores, info.num_subcores
    n_workers = num_cores * num_subcores
    tok_per_worker = n_tokens // n_workers
    C = 128
    n_chunks = tok_per_worker // C
    n_col = emb_dim // _LANES
    row_bytes = emb_dim * 4
    inv128 = 1.0 / (4 * n_pos)
    w_word = 0.25 - inv128

    mesh = plsc.VectorSubcoreMesh(core_axis_name="c", subcore_axis_name="s")

    @functools.partial(
        pl.kernel,
        out_type=jax.ShapeDtypeStruct((n_tokens, emb_dim), jnp.float32),
        mesh=mesh,
        compiler_params=pltpu.CompilerParams(use_tc_tiling_on_sc=False),
        scratch_types=[
            pltpu.VMEM((2, n_rows, C), jnp.int32),
            pltpu.VMEM((tok_per_worker, emb_dim), jnp.float32),
            pltpu.VMEM((tok_per_worker, emb_dim), jnp.float32),
            pltpu.VMEM((C, emb_dim), jnp.float32),
            pltpu.SemaphoreType.DMA,
            pltpu.SemaphoreType.DMA,
        ],
    )
    def sc_kernel(idx_hbm, word_hbm, postag_hbm, out_hbm,
                  idx_v, acc_v, wrow_v, outb_v, sem_i, sem_a):
        wid = lax.axis_index("s") * num_cores + lax.axis_index("c")
        base = wid * tok_per_worker
        blk0 = wid * n_chunks

        for ci in range(n_chunks):
            buf = ci % 2
            pltpu.sync_copy(idx_hbm.at[blk0 + ci], idx_v.at[buf])
            tok = ci * C
            cp_a = pltpu.async_copy(word_hbm.at[idx_v.at[buf, n_rows - 1]],
                                    acc_v.at[pl.ds(tok, C)], sem_i)
            cp_w = pltpu.async_copy(word_hbm.at[idx_v.at[buf, n_rows - 1]],
                                    wrow_v.at[pl.ds(tok, C)], sem_i)
            cp_a.wait()
            cp_w.wait()

            def morph_body(r, _):
                pltpu.async_copy(word_hbm.at[idx_v.at[buf, r]],
                                 acc_v.at[pl.ds(tok, C)], sem_a, add=True)
                return 0

            def pos_body(r, _):
                pltpu.async_copy(postag_hbm.at[idx_v.at[buf, r]],
                                 acc_v.at[pl.ds(tok, C)], sem_a, add=True)
                return 0

            lax.fori_loop(0, n_morph2, morph_body, 0)
            lax.fori_loop(n_morph2, n_morph2 + n_pos, pos_body, 0)

        def drain_body(_, __):
            pltpu.make_async_copy(word_hbm.at[idx_v.at[0, 0]],
                                  acc_v.at[pl.ds(0, C)], sem_a).wait()
            return 0

        lax.fori_loop(0, n_chunks * (n_morph2 + n_pos), drain_body, 0)

        for ci in range(n_chunks):
            tok = ci * C

            def out_body(t, _):
                for c in range(n_col):
                    sl = pl.ds(c * _LANES, _LANES)
                    a = acc_v[tok + t, sl]
                    w = wrow_v[tok + t, sl]
                    outb_v[t, sl] = a * inv128 + w * w_word
                return 0

            lax.fori_loop(0, C, out_body, 0)
            pltpu.sync_copy(outb_v, out_hbm.at[pl.ds(base + tok, C)])

    return sc_kernel


def kernel(word_idx, form_idx, lemma_idx, postag_idx, word_table, postag_table):
    n = word_idx.shape[0]
    n_morph = form_idx.shape[1] * form_idx.shape[2]
    emb_dim = word_table.shape[1]
    n_rows = 2 * n_morph + n_morph + 1
    C = 128

    combined = jnp.concatenate(
        [
            form_idx.reshape(n, n_morph).astype(jnp.int32),
            lemma_idx.reshape(n, n_morph).astype(jnp.int32),
            postag_idx.reshape(n, n_morph).astype(jnp.int32),
            word_idx.astype(jnp.int32)[:, None],
        ],
        axis=1,
    )
    blocks = combined.reshape(n // C, C, n_rows).transpose(0, 2, 1)

    sc = _build_sc_kernel(n, n_rows, 2 * n_morph, n_morph, emb_dim)
    return sc(blocks, word_table.astype(jnp.float32),
              postag_table.astype(jnp.float32))


# speedup vs baseline: 3.8070x; 1.3590x over previous
import functools

import jax
import jax.numpy as jnp
from jax import lax
from jax.experimental import pallas as pl
from jax.experimental.pallas import tpu as pltpu
from jax.experimental.pallas import tpu_sc as plsc

_LANES = 16


@functools.lru_cache(maxsize=None)
def _build_sc_kernel(n_tokens, n_rows, n_morph2, n_pos, n_ptab, emb_dim):
    info = plsc.get_sparse_core_info()
    num_cores, num_subcores = info.num_cores, info.num_subcores
    n_workers = num_cores * num_subcores
    tok_per_worker = n_tokens // n_workers
    C = 128
    n_chunks = tok_per_worker // C
    n_col = emb_dim // _LANES
    row_bytes = emb_dim * 4
    inv128 = 1.0 / (4 * n_pos)
    inv4 = 0.25

    mesh = plsc.VectorSubcoreMesh(core_axis_name="c", subcore_axis_name="s")

    @functools.partial(
        pl.kernel,
        out_type=jax.ShapeDtypeStruct((n_tokens, emb_dim), jnp.float32),
        mesh=mesh,
        compiler_params=pltpu.CompilerParams(use_tc_tiling_on_sc=False),
        scratch_types=[
            pltpu.VMEM((2, n_rows, C), jnp.int32),
            pltpu.VMEM((2, C, n_pos), jnp.int32),
            pltpu.VMEM((n_ptab, emb_dim), jnp.float32),
            pltpu.VMEM((tok_per_worker, emb_dim), jnp.float32),
            pltpu.VMEM((tok_per_worker, emb_dim), jnp.float32),
            pltpu.VMEM((C, emb_dim), jnp.float32),
            pltpu.SemaphoreType.DMA,
            pltpu.SemaphoreType.DMA,
        ],
    )
    def sc_kernel(idx_hbm, pidx_hbm, word_hbm, postag_hbm, out_hbm,
                  idx_v, pidx_v, ptab_v, acc_v, wrow_v, outb_v, sem_i, sem_a):
        wid = lax.axis_index("s") * num_cores + lax.axis_index("c")
        base = wid * tok_per_worker
        blk0 = wid * n_chunks

        pltpu.sync_copy(postag_hbm, ptab_v)

        wrow_cps = []
        for ci in range(n_chunks):
            buf = ci % 2
            pltpu.sync_copy(idx_hbm.at[blk0 + ci], idx_v.at[buf])
            pltpu.sync_copy(pidx_hbm.at[pl.ds(base + ci * C, C)],
                            pidx_v.at[buf])
            tok = ci * C

            def pos_body(t, _):
                pvecs = [pidx_v[buf, t, pl.ds(g * _LANES, _LANES)]
                         for g in range(n_pos // _LANES)]
                accs = [jnp.zeros((_LANES,), jnp.float32)] * n_col
                for r in range(n_pos):
                    p = pvecs[r // _LANES][r % _LANES]
                    accs = [accs[c] + ptab_v[p, pl.ds(c * _LANES, _LANES)]
                            for c in range(n_col)]
                for c in range(n_col):
                    acc_v[tok + t, pl.ds(c * _LANES, _LANES)] = accs[c]
                return 0

            lax.fori_loop(0, C, pos_body, 0)

            wrow_cps.append(
                pltpu.async_copy(word_hbm.at[idx_v.at[buf, n_rows - 1]],
                                 wrow_v.at[pl.ds(tok, C)], sem_i))

            def morph_body(r, _):
                pltpu.async_copy(word_hbm.at[idx_v.at[buf, r]],
                                 acc_v.at[pl.ds(tok, C)], sem_a, add=True)
                return 0

            lax.fori_loop(0, n_morph2, morph_body, 0)

        def drain_body(_, __):
            pltpu.make_async_copy(word_hbm.at[idx_v.at[0, 0]],
                                  acc_v.at[pl.ds(0, C)], sem_a).wait()
            return 0

        lax.fori_loop(0, n_chunks * n_morph2, drain_body, 0)
        for cp in wrow_cps:
            cp.wait()

        for ci in range(n_chunks):
            tok = ci * C

            def out_body(t, _):
                for c in range(n_col):
                    sl = pl.ds(c * _LANES, _LANES)
                    outb_v[t, sl] = (acc_v[tok + t, sl] * inv128
                                     + wrow_v[tok + t, sl] * inv4)
                return 0

            lax.fori_loop(0, C, out_body, 0)
            pltpu.sync_copy(outb_v, out_hbm.at[pl.ds(base + tok, C)])

    return sc_kernel


def kernel(word_idx, form_idx, lemma_idx, postag_idx, word_table, postag_table):
    n = word_idx.shape[0]
    n_morph = form_idx.shape[1] * form_idx.shape[2]
    emb_dim = word_table.shape[1]
    n_rows = 2 * n_morph + 1
    C = 128

    combined = jnp.concatenate(
        [
            form_idx.reshape(n, n_morph).astype(jnp.int32),
            lemma_idx.reshape(n, n_morph).astype(jnp.int32),
            word_idx.astype(jnp.int32)[:, None],
        ],
        axis=1,
    )
    blocks = combined.reshape(n // C, C, n_rows).transpose(0, 2, 1)
    pidx = postag_idx.reshape(n, n_morph).astype(jnp.int32)

    sc = _build_sc_kernel(n, n_rows, 2 * n_morph, n_morph,
                          postag_table.shape[0], emb_dim)
    return sc(blocks, pidx, word_table.astype(jnp.float32),
              postag_table.astype(jnp.float32))
